# baseline (speedup 1.0000x reference)
import jax
import jax.numpy as jnp
from jax.experimental import pallas as pl
from jax.experimental.pallas import tpu as pltpu

_B, _T, _D, _K = 1024, 32, 256, 256
_N = _B * _T
_BLK = 8192
_NBLK = _N // _BLK


def _vq_body(z_ref, cb_ref, out_ref, idx_ref, loss_ref, csq_ref):
    cb = cb_ref[...]

    @pl.when(pl.program_id(0) == 0)
    def _csq():
        csq_col = jnp.sum(cb * cb, axis=1, keepdims=True)
        ii = jax.lax.broadcasted_iota(jnp.int32, (_K, _K), 0)
        jj = jax.lax.broadcasted_iota(jnp.int32, (_K, _K), 1)
        eye = (ii == jj).astype(jnp.float32)
        csq_ref[...] = jax.lax.dot_general(
            csq_col, eye, (((0,), (0,)), ((), ())),
            preferred_element_type=jnp.float32,
            precision=jax.lax.Precision.HIGHEST)

    z = z_ref[...]
    scores = jax.lax.dot_general(
        z, cb, (((1,), (1,)), ((), ())),
        preferred_element_type=jnp.float32)
    z_sq = jnp.sum(z * z, axis=1, keepdims=True)
    dist = (z_sq - 2.0 * scores) + csq_ref[...]
    m = jnp.min(dist, axis=1, keepdims=True)
    iota = jax.lax.broadcasted_iota(jnp.int32, dist.shape, 1)
    pick = jnp.where(dist == m, iota, _K)
    idx = jnp.min(pick, axis=1, keepdims=True)
    idx_ref[...] = idx
    onehot = (iota == idx).astype(jnp.float32)
    quant = jax.lax.dot_general(
        onehot, cb, (((1,), (0,)), ((), ())),
        preferred_element_type=jnp.float32)
    out_ref[...] = z + (quant - z)
    diff = z - quant
    psum = jnp.sum(diff * diff, axis=(0, 1), keepdims=True)

    @pl.when(pl.program_id(0) == 0)
    def _init():
        loss_ref[...] = psum

    @pl.when(pl.program_id(0) != 0)
    def _acc():
        loss_ref[...] += psum

    @pl.when(pl.program_id(0) == _NBLK - 1)
    def _finish():
        loss_ref[...] = loss_ref[...] * (1.25 / float(_N * _D))


def kernel(z, codebook):
    zf = z.reshape(_N, _D)
    out, idx, loss = pl.pallas_call(
        _vq_body,
        grid=(_NBLK,),
        in_specs=[
            pl.BlockSpec((_BLK, _D), lambda i: (i, 0)),
            pl.BlockSpec((_K, _D), lambda i: (0, 0)),
        ],
        out_specs=[
            pl.BlockSpec((_BLK, _D), lambda i: (i, 0)),
            pl.BlockSpec((_BLK, 1), lambda i: (i, 0)),
            pl.BlockSpec((1, 1), lambda i: (0, 0)),
        ],
        out_shape=[
            jax.ShapeDtypeStruct((_N, _D), jnp.float32),
            jax.ShapeDtypeStruct((_N, 1), jnp.int32),
            jax.ShapeDtypeStruct((1, 1), jnp.float32),
        ],
        scratch_shapes=[pltpu.VMEM((1, _K), jnp.float32)],
    )(zf, codebook)
    return (out.reshape(_B, _T, _D),
            idx.reshape(_B, _T),
            loss[0, 0])


# speedup vs baseline: 1.0743x; 1.0743x over previous
import jax
import jax.numpy as jnp
from jax.experimental import pallas as pl
from jax.experimental.pallas import tpu as pltpu

_B, _T, _D, _K = 1024, 32, 256, 256
_N = _B * _T
_BLK = 8192
_NBLK = _N // _BLK


def _vq_body(z_ref, cb_ref, out_ref, idx_ref, loss_ref, csq_ref):
    cb = cb_ref[...]

    @pl.when(pl.program_id(0) == 0)
    def _csq():
        csq_col = jnp.sum(cb * cb, axis=1, keepdims=True)
        ii = jax.lax.broadcasted_iota(jnp.int32, (_K, _K), 0)
        jj = jax.lax.broadcasted_iota(jnp.int32, (_K, _K), 1)
        eye = (ii == jj).astype(jnp.float32)
        csq_ref[...] = jax.lax.dot_general(
            csq_col, eye, (((0,), (0,)), ((), ())),
            preferred_element_type=jnp.float32,
            precision=jax.lax.Precision.HIGHEST)

    z = z_ref[...]
    scores = jax.lax.dot_general(
        z, cb, (((1,), (1,)), ((), ())),
        preferred_element_type=jnp.float32)
    dist = csq_ref[...] - 2.0 * scores
    m = jnp.min(dist, axis=1, keepdims=True)
    iota = jax.lax.broadcasted_iota(jnp.int32, dist.shape, 1)
    pick = jnp.where(dist == m, iota, _K)
    idx = jnp.min(pick, axis=1, keepdims=True)
    idx_ref[...] = idx
    onehot = (iota == idx).astype(jnp.float32)
    quant = jax.lax.dot_general(
        onehot, cb, (((1,), (0,)), ((), ())),
        preferred_element_type=jnp.float32)
    out_ref[...] = z + (quant - z)
    diff = z - quant
    psum = jnp.sum(diff * diff, axis=(0, 1), keepdims=True)

    @pl.when(pl.program_id(0) == 0)
    def _init():
        loss_ref[...] = psum

    @pl.when(pl.program_id(0) != 0)
    def _acc():
        loss_ref[...] += psum

    @pl.when(pl.program_id(0) == _NBLK - 1)
    def _finish():
        loss_ref[...] = loss_ref[...] * (1.25 / float(_N * _D))


def kernel(z, codebook):
    zf = z.reshape(_N, _D)
    out, idx, loss = pl.pallas_call(
        _vq_body,
        grid=(_NBLK,),
        in_specs=[
            pl.BlockSpec((_BLK, _D), lambda i: (i, 0)),
            pl.BlockSpec((_K, _D), lambda i: (0, 0)),
        ],
        out_specs=[
            pl.BlockSpec((_BLK, _D), lambda i: (i, 0)),
            pl.BlockSpec((_BLK, 1), lambda i: (i, 0)),
            pl.BlockSpec((1, 1), lambda i: (0, 0)),
        ],
        out_shape=[
            jax.ShapeDtypeStruct((_N, _D), jnp.float32),
            jax.ShapeDtypeStruct((_N, 1), jnp.int32),
            jax.ShapeDtypeStruct((1, 1), jnp.float32),
        ],
        scratch_shapes=[pltpu.VMEM((1, _K), jnp.float32)],
    )(zf, codebook)
    return (out.reshape(_B, _T, _D),
            idx.reshape(_B, _T),
            loss[0, 0])
